# Initial kernel scaffold; baseline (speedup 1.0000x reference)
import jax
import jax.numpy as jnp
from jax.experimental import pallas as pl


def kernel(x_ncl, w1c, b1c, w2c, b2c, w3c, b3c, w4c, b4c, wf1, bf1, wf2, bf2):
    raise NotImplementedError("write your pallas kernel here")


# speedup vs baseline: 1.0100x; 1.0100x over previous
import jax
import jax.numpy as jnp
from jax.experimental import pallas as pl
from jax.experimental.pallas import tpu as pltpu

C = 128
K = 4
R0 = 8


def _fused_kernel(x_ref, w1_ref, b1_ref, w2_ref, b2_ref, w3_ref, b3_ref,
                  w4_ref, b4_ref, wf1_ref, bf1_ref, wf2_ref, bf2_ref,
                  o_ref, xpad_ref, col_ref, flat_ref):
    Bb, L, _ = x_ref.shape
    f32, bf16 = jnp.float32, jnp.bfloat16

    xpad_ref[:, R0 - 1:R0, :] = jnp.zeros((Bb, 1, C), bf16)
    xpad_ref[:, R0 + L:R0 + L + 2, :] = jnp.zeros((Bb, 2, C), bf16)
    xpad_ref[:, R0:R0 + L, :] = (
        x_ref[...] * jnp.ones((1, 1, C), f32)).astype(bf16)
    acc = xpad_ref[:, R0 - 1:R0 - 1 + L, :].astype(f32) * w1_ref[0]
    for j in range(1, K):
        acc = acc + xpad_ref[:, R0 - 1 + j:R0 - 1 + j + L, :].astype(f32) * w1_ref[j]
    h = jnp.maximum(acc + b1_ref[...], 0.0)

    col_ref[:, R0:R0 + 1, 0:C] = jnp.zeros((Bb, 1, C), bf16)
    col_ref[:, R0 + L - 1:R0 + L, 2 * C:3 * C] = jnp.zeros((Bb, 1, C), bf16)
    col_ref[:, R0 + L - 2:R0 + L, 3 * C:4 * C] = jnp.zeros((Bb, 2, C), bf16)

    def conv_block(h, w_ref, b_ref):
        hb = h.astype(bf16)
        for j in range(K):
            col_ref[:, R0 + 1 - j:R0 + 1 - j + L, j * C:(j + 1) * C] = hb
        y = jax.lax.dot_general(
            col_ref[:, R0:R0 + L, :], w_ref[...],
            dimension_numbers=(((2,), (0,)), ((), ())),
            preferred_element_type=f32)
        return jnp.maximum(y + b_ref[...], 0.0)

    h = conv_block(h, w2_ref, b2_ref)
    h = conv_block(h, w3_ref, b3_ref)
    h = conv_block(h, w4_ref, b4_ref)

    hb = h.astype(bf16)
    for t in range(L):
        flat_ref[:, t * C:(t + 1) * C] = hb[:, t, :]
    z = jnp.dot(flat_ref[...], wf1_ref[...],
                preferred_element_type=f32) + bf1_ref[...]
    z = jnp.maximum(z, 0.0)
    o_ref[...] = jnp.sum(z * wf2_ref[...], axis=-1, keepdims=True) + bf2_ref[...]


def kernel(x_ncl, w1c, b1c, w2c, b2c, w3c, b3c, w4c, b4c, wf1, bf1, wf2, bf2):
    B, cin0, L = x_ncl.shape
    x_blc = jnp.transpose(x_ncl, (0, 2, 1)).astype(jnp.float32)

    bb = 32
    grid = (B // bb,)

    bf16 = jnp.bfloat16
    weights = [w1c, b1c, w2c.astype(bf16), b2c, w3c.astype(bf16), b3c,
               w4c.astype(bf16), b4c, wf1.astype(bf16), bf1, wf2, bf2]

    def rep_spec(shape):
        n = len(shape)
        return pl.BlockSpec(shape, lambda i, n=n: (0,) * n)

    in_specs = ([pl.BlockSpec((bb, L, 1), lambda i: (i, 0, 0))]
                + [rep_spec(w.shape) for w in weights])
    out_specs = pl.BlockSpec((bb, 1), lambda i: (i, 0))

    return pl.pallas_call(
        _fused_kernel,
        out_shape=jax.ShapeDtypeStruct((B, 1), jnp.float32),
        grid=grid,
        in_specs=in_specs,
        out_specs=out_specs,
        scratch_shapes=[
            pltpu.VMEM((bb, L + 16, C), bf16),
            pltpu.VMEM((bb, L + 16, K * C), bf16),
            pltpu.VMEM((bb, L * C), bf16),
        ],
        compiler_params=pltpu.CompilerParams(dimension_semantics=("parallel",)),
    )(x_blc, *weights)


# speedup vs baseline: 1.0198x; 1.0097x over previous
import jax
import jax.numpy as jnp
from jax.experimental import pallas as pl
from jax.experimental.pallas import tpu as pltpu

C = 128
K = 4
R0 = 8


def _fused_kernel(x_ref, w1_ref, b1_ref, w2_ref, b2_ref, w3_ref, b3_ref,
                  w4_ref, b4_ref, wf1_ref, bf1_ref, wf2_ref, bf2_ref,
                  o_ref, col_ref, flat_ref):
    Bb, L, _ = x_ref.shape
    f32, bf16 = jnp.float32, jnp.bfloat16

    col_ref[:, R0:R0 + 1, 0:C] = jnp.zeros((Bb, 1, C), bf16)
    col_ref[:, R0 + L - 1:R0 + L, 2 * C:3 * C] = jnp.zeros((Bb, 1, C), bf16)
    col_ref[:, R0 + L - 2:R0 + L, 3 * C:4 * C] = jnp.zeros((Bb, 2, C), bf16)

    def conv_block(h, w_ref, b_ref):
        hb = h.astype(bf16)
        for j in range(K):
            col_ref[:, R0 + 1 - j:R0 + 1 - j + L, j * C:(j + 1) * C] = hb
        y = jax.lax.dot_general(
            col_ref[:, R0:R0 + L, :], w_ref[...],
            dimension_numbers=(((2,), (0,)), ((), ())),
            preferred_element_type=f32)
        return jnp.maximum(y + b_ref[...], 0.0)

    xb = jax.lax.dot_general(
        x_ref[...].astype(bf16), jnp.ones((1, C), bf16),
        dimension_numbers=(((2,), (0,)), ((), ())),
        preferred_element_type=f32)
    h = conv_block(xb, w1_ref, b1_ref)
    h = conv_block(h, w2_ref, b2_ref)
    h = conv_block(h, w3_ref, b3_ref)
    h = conv_block(h, w4_ref, b4_ref)

    hb = h.astype(bf16)
    for t in range(L):
        flat_ref[:, t * C:(t + 1) * C] = hb[:, t, :]
    z = jnp.dot(flat_ref[...], wf1_ref[...],
                preferred_element_type=f32) + bf1_ref[...]
    z = jnp.maximum(z, 0.0)
    o_ref[...] = jnp.sum(z * wf2_ref[...], axis=-1, keepdims=True) + bf2_ref[...]


def kernel(x_ncl, w1c, b1c, w2c, b2c, w3c, b3c, w4c, b4c, wf1, bf1, wf2, bf2):
    B, cin0, L = x_ncl.shape
    x_blc = jnp.transpose(x_ncl, (0, 2, 1)).astype(jnp.float32)

    bb = 32
    grid = (B // bb,)

    bf16 = jnp.bfloat16
    w1e = jnp.pad(w1c, ((0, 0), (0, C - 1), (0, 0))).reshape(K * C, C)
    weights = [w1e.astype(bf16), b1c, w2c.astype(bf16), b2c, w3c.astype(bf16),
               b3c, w4c.astype(bf16), b4c, wf1.astype(bf16), bf1, wf2, bf2]

    def rep_spec(shape):
        n = len(shape)
        return pl.BlockSpec(shape, lambda i, n=n: (0,) * n)

    in_specs = ([pl.BlockSpec((bb, L, 1), lambda i: (i, 0, 0))]
                + [rep_spec(w.shape) for w in weights])
    out_specs = pl.BlockSpec((bb, 1), lambda i: (i, 0))

    return pl.pallas_call(
        _fused_kernel,
        out_shape=jax.ShapeDtypeStruct((B, 1), jnp.float32),
        grid=grid,
        in_specs=in_specs,
        out_specs=out_specs,
        scratch_shapes=[
            pltpu.VMEM((bb, L + 16, K * C), bf16),
            pltpu.VMEM((bb, L * C), bf16),
        ],
        compiler_params=pltpu.CompilerParams(dimension_semantics=("parallel",)),
    )(x_blc, *weights)
